# baseline (speedup 1.0000x reference)
import functools

import jax
import jax.numpy as jnp
from jax import lax
from jax.experimental import pallas as pl
from jax.experimental.pallas import tpu as pltpu
from jax.experimental.pallas import tpu_sc as plsc

_RADIUS2 = 0.009999999776482582
_K = 32

_SBLK = 256
_D = 144
_CHUNK = 512
_GSUB = 128


def _ballq_body(q_ref, x_ref, p_ref, idx_ref, *, n_points):
    n = n_points
    nw = n // 16
    b = pl.program_id(0)
    qx = q_ref[0, :, 0:1]
    qy = q_ref[0, :, 1:2]
    qz = q_ref[0, :, 2:3]
    xx = x_ref[0, 0:1, :]
    xy = x_ref[0, 1:2, :]
    xz = x_ref[0, 2:3, :]
    dx = qx - xx
    dy = qy - xy
    dz = qz - xz
    d2 = (dx * dx + dy * dy) + dz * dz
    mask = jnp.where(d2 < _RADIUS2, 1.0, 0.0)
    words = jax.lax.dot_general(
        mask, p_ref[...], (((1,), (0,)), ((), ())),
        precision=jax.lax.Precision.DEFAULT,
        preferred_element_type=jnp.float32).astype(jnp.int32)
    wiota = jax.lax.broadcasted_iota(jnp.int32, (_SBLK, nw), 1)
    kiota = jax.lax.broadcasted_iota(jnp.int32, (_SBLK, _K), 1)
    idxm = jnp.full((_SBLK, _K), n, jnp.int32)
    for k in range(_K):
        a = jnp.min(jnp.where(words != 0, wiota, nw), axis=1, keepdims=True)
        eq = wiota == a
        wv = jnp.sum(jnp.where(eq, words, 0), axis=1, keepdims=True)
        iso = wv & -wv
        e = (jax.lax.bitcast_convert_type(iso.astype(jnp.float32),
                                          jnp.int32) >> 23) - 127
        idxk = jnp.where(a < nw, a * 16 + e, n)
        idxm = jnp.where(kiota == k, idxk, idxm)
        if k + 1 < _K:
            words = jnp.where(eq, wv & (wv - 1), words)
    first = idxm[:, 0:1]
    first = jnp.where(first < n, first, 0)
    idx_ref[0] = jnp.where(idxm < n, idxm, first) + b * n


def _sc_gather(tab_hbm, idx_hbm, qsub_hbm, out_hbm, idx_v, rows_v, qs_v, sem):
    info = plsc.get_sparse_core_info()
    nw = info.num_cores * info.num_subcores
    wid = lax.axis_index("s") * info.num_cores + lax.axis_index("c")
    n_chunks = out_hbm.shape[0] // nw
    lanes = jax.lax.broadcasted_iota(jnp.int32, (16,), 0)

    def chunk_body(ch, carry):
        cid = wid * n_chunks + ch
        pltpu.sync_copy(idx_hbm.at[cid], idx_v)
        copies = [
            pltpu.async_copy(
                tab_hbm.at[idx_v.at[i]],
                rows_v.at[pl.ds(i * _GSUB, _GSUB)],
                sem,
            )
            for i in range(_CHUNK // _GSUB)
        ]
        pltpu.sync_copy(qsub_hbm.at[cid], qs_v)
        for c in copies:
            c.wait()
        def row_body(r, rcarry):
            qv = qs_v[lax.shift_right_logical(r, 5), pl.ds(0, 16)]
            rows_v[r, pl.ds(0, 16)] = rows_v[r, pl.ds(0, 16)] - qv
            return rcarry

        lax.fori_loop(0, _CHUNK, row_body, 0)
        pltpu.sync_copy(rows_v, out_hbm.at[cid])
        return carry

    lax.fori_loop(0, n_chunks, chunk_body, 0)


def _half(xyz, new_xyz, features, packp):
    B, N, _ = xyz.shape
    S = new_xyz.shape[1]
    C = features.shape[1]
    NC = C + 3
    BSK = B * S * _K

    xyz_t = jnp.transpose(xyz, (0, 2, 1))

    idx = pl.pallas_call(
        functools.partial(_ballq_body, n_points=N),
        grid=(B, S // _SBLK),
        in_specs=[
            pl.BlockSpec((1, _SBLK, 3), lambda b, s: (b, s, 0)),
            pl.BlockSpec((1, 3, N), lambda b, s: (b, 0, 0)),
            pl.BlockSpec((N, N // 16), lambda b, s: (0, 0)),
        ],
        out_specs=pl.BlockSpec((1, _SBLK, _K), lambda b, s: (b, s, 0)),
        out_shape=jax.ShapeDtypeStruct((B, S, _K), jnp.int32),
        compiler_params=pltpu.CompilerParams(
            dimension_semantics=("parallel", "arbitrary")),
    )(new_xyz, xyz_t, packp)

    tab = jnp.concatenate(
        [xyz, jnp.transpose(features, (0, 2, 1)),
         jnp.zeros((B, N, _D - NC), jnp.float32)], axis=2).reshape(B * N, _D)
    idx3d = idx.reshape(BSK // _CHUNK, _CHUNK // _GSUB, _GSUB)
    qsub = jnp.pad(new_xyz.reshape(B * S, 3),
                   ((0, 0), (0, 13))).reshape(BSK // _CHUNK, _CHUNK // _K, 16)

    mesh = plsc.VectorSubcoreMesh(core_axis_name="c", subcore_axis_name="s")
    rows = pl.kernel(
        _sc_gather,
        mesh=mesh,
        out_type=jax.ShapeDtypeStruct((BSK // _CHUNK, _CHUNK, _D), jnp.float32),
        scratch_types=[
            pltpu.VMEM((_CHUNK // _GSUB, _GSUB), jnp.int32),
            pltpu.VMEM((_CHUNK, _D), jnp.float32),
            pltpu.VMEM((_CHUNK // _K, 16), jnp.float32),
            pltpu.SemaphoreType.DMA,
        ],
        compiler_params=pltpu.CompilerParams(use_tc_tiling_on_sc=False),
    )(tab, idx3d, qsub)

    out = jnp.transpose(rows.reshape(B, S * _K, _D), (0, 2, 1))[:, :NC, :]
    return out.reshape(B, NC, S, _K)


def kernel(xyz, new_xyz, features):
    B, N, _ = xyz.shape

    jj = jnp.arange(N, dtype=jnp.int32)
    packp = jnp.where((jj[:, None] >> 4) == jnp.arange(N // 16,
                                                       dtype=jnp.int32)[None, :],
                      (1 << (jj[:, None] & 15)).astype(jnp.float32), 0.0)

    h = B // 2
    out0 = _half(xyz[:h], new_xyz[:h], features[:h], packp)
    out1 = _half(xyz[h:], new_xyz[h:], features[h:], packp)
    return jnp.concatenate([out0, out1], axis=0)


# speedup vs baseline: 1.0664x; 1.0664x over previous
import functools

import jax
import jax.numpy as jnp
from jax import lax
from jax.experimental import pallas as pl
from jax.experimental.pallas import tpu as pltpu
from jax.experimental.pallas import tpu_sc as plsc

_RADIUS2 = 0.009999999776482582
_K = 32

_SBLK = 256
_D = 144
_CHUNK = 256
_GSUB = 128


def _ballq_body(q_ref, x_ref, p_ref, idx_ref, *, n_points):
    n = n_points
    nw = n // 16
    b = pl.program_id(0)
    qx = q_ref[0, :, 0:1]
    qy = q_ref[0, :, 1:2]
    qz = q_ref[0, :, 2:3]
    xx = x_ref[0, 0:1, :]
    xy = x_ref[0, 1:2, :]
    xz = x_ref[0, 2:3, :]
    dx = qx - xx
    dy = qy - xy
    dz = qz - xz
    d2 = (dx * dx + dy * dy) + dz * dz
    mask = jnp.where(d2 < _RADIUS2, 1.0, 0.0)
    words = jax.lax.dot_general(
        mask, p_ref[...], (((1,), (0,)), ((), ())),
        precision=jax.lax.Precision.DEFAULT,
        preferred_element_type=jnp.float32).astype(jnp.int32)
    wiota = jax.lax.broadcasted_iota(jnp.int32, (_SBLK, nw), 1)
    kiota = jax.lax.broadcasted_iota(jnp.int32, (_SBLK, _K), 1)
    idxm = jnp.full((_SBLK, _K), n, jnp.int32)
    for k in range(_K):
        a = jnp.min(jnp.where(words != 0, wiota, nw), axis=1, keepdims=True)
        eq = wiota == a
        wv = jnp.sum(jnp.where(eq, words, 0), axis=1, keepdims=True)
        iso = wv & -wv
        e = (jax.lax.bitcast_convert_type(iso.astype(jnp.float32),
                                          jnp.int32) >> 23) - 127
        idxk = jnp.where(a < nw, a * 16 + e, n)
        idxm = jnp.where(kiota == k, idxk, idxm)
        if k + 1 < _K:
            words = jnp.where(eq, wv & (wv - 1), words)
    first = idxm[:, 0:1]
    first = jnp.where(first < n, first, 0)
    idx_ref[0] = jnp.where(idxm < n, idxm, first) + b * n


def _sc_gather(tab_hbm, idx_hbm, qsub_hbm, out_hbm,
               idx_v0, idx_v1, rows_v0, rows_v1, qs_v0, qs_v1,
               gsem0, gsem1, osem0, osem1):
    idx_v = [idx_v0, idx_v1]
    rows_v = [rows_v0, rows_v1]
    qs_v = [qs_v0, qs_v1]
    gsem = [gsem0, gsem1]
    osem = [osem0, osem1]
    info = plsc.get_sparse_core_info()
    nw = info.num_cores * info.num_subcores
    wid = lax.axis_index("s") * info.num_cores + lax.axis_index("c")
    n_chunks = out_hbm.shape[0] // nw

    def fire(b, ch):
        cid = wid * n_chunks + ch
        pltpu.sync_copy(idx_hbm.at[cid], idx_v[b])
        copies = [
            pltpu.async_copy(
                tab_hbm.at[idx_v[b].at[i]],
                rows_v[b].at[pl.ds(i * _GSUB, _GSUB)],
                gsem[b],
            )
            for i in range(_CHUNK // _GSUB)
        ]
        pltpu.sync_copy(qsub_hbm.at[cid], qs_v[b])
        return copies

    def subtract(b):
        def row_body(r, rcarry):
            qv = qs_v[b][lax.shift_right_logical(r, 5), pl.ds(0, 16)]
            rows_v[b][r, pl.ds(0, 16)] = rows_v[b][r, pl.ds(0, 16)] - qv
            return rcarry

        lax.fori_loop(0, _CHUNK, row_body, 0)

    gath = [None, None]
    outc = [None, None]
    for ch in range(n_chunks):
        b = ch % 2
        if outc[b] is not None:
            outc[b].wait()
        gath[b] = fire(b, ch)
        pb = 1 - b
        if gath[pb] is not None:
            for c in gath[pb]:
                c.wait()
            subtract(pb)
            outc[pb] = pltpu.async_copy(
                rows_v[pb], out_hbm.at[wid * n_chunks + ch - 1], osem[pb])
            gath[pb] = None
    b = (n_chunks - 1) % 2
    for c in gath[b]:
        c.wait()
    subtract(b)
    pltpu.sync_copy(rows_v[b], out_hbm.at[wid * n_chunks + n_chunks - 1])
    if outc[1 - b] is not None:
        outc[1 - b].wait()


def _half(xyz, new_xyz, features, packp):
    B, N, _ = xyz.shape
    S = new_xyz.shape[1]
    C = features.shape[1]
    NC = C + 3
    BSK = B * S * _K

    xyz_t = jnp.transpose(xyz, (0, 2, 1))

    idx = pl.pallas_call(
        functools.partial(_ballq_body, n_points=N),
        grid=(B, S // _SBLK),
        in_specs=[
            pl.BlockSpec((1, _SBLK, 3), lambda b, s: (b, s, 0)),
            pl.BlockSpec((1, 3, N), lambda b, s: (b, 0, 0)),
            pl.BlockSpec((N, N // 16), lambda b, s: (0, 0)),
        ],
        out_specs=pl.BlockSpec((1, _SBLK, _K), lambda b, s: (b, s, 0)),
        out_shape=jax.ShapeDtypeStruct((B, S, _K), jnp.int32),
        compiler_params=pltpu.CompilerParams(
            dimension_semantics=("parallel", "arbitrary")),
    )(new_xyz, xyz_t, packp)

    tab = jnp.concatenate(
        [xyz, jnp.transpose(features, (0, 2, 1)),
         jnp.zeros((B, N, _D - NC), jnp.float32)], axis=2).reshape(B * N, _D)
    idx3d = idx.reshape(BSK // _CHUNK, _CHUNK // _GSUB, _GSUB)
    qsub = jnp.pad(new_xyz.reshape(B * S, 3),
                   ((0, 0), (0, 13))).reshape(BSK // _CHUNK, _CHUNK // _K, 16)

    mesh = plsc.VectorSubcoreMesh(core_axis_name="c", subcore_axis_name="s")
    rows = pl.kernel(
        _sc_gather,
        mesh=mesh,
        out_type=jax.ShapeDtypeStruct((BSK // _CHUNK, _CHUNK, _D), jnp.float32),
        scratch_types=[
            pltpu.VMEM((_CHUNK // _GSUB, _GSUB), jnp.int32),
            pltpu.VMEM((_CHUNK // _GSUB, _GSUB), jnp.int32),
            pltpu.VMEM((_CHUNK, _D), jnp.float32),
            pltpu.VMEM((_CHUNK, _D), jnp.float32),
            pltpu.VMEM((_CHUNK // _K, 16), jnp.float32),
            pltpu.VMEM((_CHUNK // _K, 16), jnp.float32),
            pltpu.SemaphoreType.DMA,
            pltpu.SemaphoreType.DMA,
            pltpu.SemaphoreType.DMA,
            pltpu.SemaphoreType.DMA,
        ],
        compiler_params=pltpu.CompilerParams(use_tc_tiling_on_sc=False),
    )(tab, idx3d, qsub)

    out = jnp.transpose(rows.reshape(B, S * _K, _D), (0, 2, 1))[:, :NC, :]
    return out.reshape(B, NC, S, _K)


def kernel(xyz, new_xyz, features):
    B, N, _ = xyz.shape

    jj = jnp.arange(N, dtype=jnp.int32)
    packp = jnp.where((jj[:, None] >> 4) == jnp.arange(N // 16,
                                                       dtype=jnp.int32)[None, :],
                      (1 << (jj[:, None] & 15)).astype(jnp.float32), 0.0)

    return _half(xyz, new_xyz, features, packp)
